# baseline (speedup 1.0000x reference)
import functools

import jax
import jax.numpy as jnp
from jax import lax
from jax.experimental import pallas as pl
from jax.experimental.pallas import tpu as pltpu
from jax.experimental.pallas import tpu_sc as plsc

N_NODES = 10000
D = 128
ROW_BLOCK = 2000

NUM_CORES = 2
NUM_TILES = 16
NUM_WORKERS = NUM_CORES * NUM_TILES
WIN = 80
NBUF = 3
AGG_ROWS = 10240
STRIPE = AGG_ROWS // NUM_TILES


def _fin_kernel(p0_ref, p1_ref, d0_ref, d1_ref, x_ref, w_ref, b_ref, o_ref):
    deg_raw = d0_ref[...] + d1_ref[...]
    deg = jnp.maximum(deg_raw, 1.0)
    s = (p0_ref[...][0] + p1_ref[...][0]) / deg + x_ref[...]
    mm = jnp.dot(s, w_ref[...], preferred_element_type=jnp.float32)
    bfac = 1.0 + (deg_raw >= 0.5).astype(jnp.float32)
    o_ref[...] = jnp.maximum(mm + b_ref[...] * bfac, 0.0)


@functools.lru_cache(maxsize=None)
def _make_sc(nl, rem):
    assert nl >= 26
    UNROLL = 12
    k_end = (nl - 2) // UNROLL
    mesh = plsc.VectorSubcoreMesh(core_axis_name="c", subcore_axis_name="s")

    @functools.partial(
        pl.kernel,
        mesh=mesh,
        out_type=[
            jax.ShapeDtypeStruct((NUM_CORES, AGG_ROWS, D), jnp.float32),
            jax.ShapeDtypeStruct((NUM_CORES, AGG_ROWS), jnp.float32),
        ],
        scratch_types=[
            pltpu.VMEM((4, WIN), jnp.int32),
            pltpu.VMEM((4, WIN), jnp.int32),
            pltpu.VMEM((NBUF, WIN, D), jnp.float32),
            pltpu.VMEM((WIN,), jnp.float32),
            pltpu.VMEM_SHARED((AGG_ROWS, D), jnp.float32),
            pltpu.VMEM_SHARED((AGG_ROWS,), jnp.float32),
        ] + [pltpu.SemaphoreType.DMA] * (4 + 2 * NBUF + 1),
    )
    def sc(x_hbm, src_hbm, dst_hbm, zrows_hbm, zdeg_hbm,
           agg_out, deg_out,
           sidx, didx, rows, ones, agg_sh, deg_sh,
           *sems):
        cid = lax.axis_index("c")
        tid = lax.axis_index("s")
        chunk = tid * NUM_CORES + cid
        isems = sems[0:4]
        rsems = sems[4:4 + NBUF]
        ssems = sems[4 + NBUF:4 + 2 * NBUF]
        zsem = sems[4 + 2 * NBUF]

        def wg(w):
            return chunk + NUM_WORKERS * w

        def i_start(w, s):
            o = wg(w) * WIN
            pltpu.async_copy(src_hbm.at[pl.ds(o, WIN)], sidx.at[s], isems[s])
            pltpu.async_copy(dst_hbm.at[pl.ds(o, WIN)], didx.at[s], isems[s])

        def i_wait(w, s):
            o = wg(w) * WIN
            pltpu.make_async_copy(src_hbm.at[pl.ds(o, WIN)], sidx.at[s], isems[s]).wait()
            pltpu.make_async_copy(dst_hbm.at[pl.ds(o, WIN)], didx.at[s], isems[s]).wait()

        def g_start(w, s, r):
            pltpu.async_copy(x_hbm.at[sidx.at[s]], rows.at[r], rsems[r])

        def g_wait(w, s, r):
            pltpu.make_async_copy(x_hbm.at[sidx.at[s]], rows.at[r], rsems[r]).wait()

        def s_start(w, s, r):
            pltpu.async_copy(rows.at[r], agg_sh.at[didx.at[s]], ssems[r], add=True)
            pltpu.async_copy(ones, deg_sh.at[didx.at[s]], ssems[r], add=True)

        def s_wait(w, s, r):
            pltpu.make_async_copy(rows.at[r], agg_sh.at[didx.at[s]], ssems[r]).wait()
            pltpu.make_async_copy(ones, deg_sh.at[didx.at[s]], ssems[r]).wait()

        def step(w, j):
            static = isinstance(w, int)
            if not static or w >= NBUF - 1:
                wp = w - (NBUF - 1)
                s_wait(wp, (j - (NBUF - 1)) % 4, (j - (NBUF - 1)) % NBUF)
            if not static or w + 2 < nl:
                i_start(w + 2, (j + 2) % 4)
            if not static or w + 1 < nl:
                i_wait(w + 1, (j + 1) % 4)
                g_start(w + 1, (j + 1) % 4, (j + 1) % NBUF)
            g_wait(w, j % 4, j % NBUF)
            s_start(w, j % 4, j % NBUF)

        for j in range(WIN // 16):
            ones[pl.ds(16 * j, 16)] = jnp.full((16,), 1.0, jnp.float32)

        zc1 = pltpu.async_copy(zrows_hbm, agg_sh.at[pl.ds(tid * STRIPE, STRIPE)], zsem)
        zc2 = pltpu.async_copy(zdeg_hbm, deg_sh.at[pl.ds(tid * STRIPE, STRIPE)], zsem)
        i_start(0, 0)
        i_start(1, 1)
        i_wait(0, 0)
        g_start(0, 0, 0)
        i_start(2, 2)
        i_wait(1, 1)
        g_start(1, 1, 1)
        g_wait(0, 0, 0)
        zc1.wait()
        zc2.wait()
        plsc.subcore_barrier()
        s_start(0, 0, 0)
        for w in range(1, UNROLL):
            step(w, w % UNROLL)

        def body(k, carry):
            w0 = UNROLL * k
            for j in range(UNROLL):
                step(w0 + j, j)
            return carry

        lax.fori_loop(1, k_end, body, 0)
        for w in range(UNROLL * k_end, nl):
            step(w, w % UNROLL)
        for w in range(nl - NBUF + 1, nl):
            s_wait(w, w % 4, w % NBUF)

        if rem:
            @pl.when(chunk < rem)
            def _():
                wr = (NUM_WORKERS * nl + chunk) * WIN
                pltpu.sync_copy(src_hbm.at[pl.ds(wr, WIN)], sidx.at[0])
                pltpu.sync_copy(dst_hbm.at[pl.ds(wr, WIN)], didx.at[0])
                pltpu.sync_copy(x_hbm.at[sidx.at[0]], rows.at[0])
                pltpu.sync_copy(rows.at[0], agg_sh.at[didx.at[0]], add=True)
                pltpu.sync_copy(ones, deg_sh.at[didx.at[0]], add=True)

        plsc.subcore_barrier()
        pltpu.sync_copy(agg_sh.at[pl.ds(tid * STRIPE, STRIPE)],
                        agg_out.at[cid, pl.ds(tid * STRIPE, STRIPE)])
        pltpu.sync_copy(deg_sh.at[pl.ds(tid * STRIPE, STRIPE)],
                        deg_out.at[cid, pl.ds(tid * STRIPE, STRIPE)])

    return sc


def kernel(x, edge_index, W1, b1):
    n, d = x.shape
    e = edge_index.shape[1]
    ei = edge_index.astype(jnp.int32)
    if e % WIN:
        padn = WIN - e % WIN
        pidx = jnp.arange(padn, dtype=jnp.int32)
        ei = jnp.concatenate(
            [ei, jnp.stack([pidx % n, n + pidx % (AGG_ROWS - n)])], axis=1)
        e += padn
    nwt = e // WIN
    src1 = ei[0]
    dst1 = ei[1]
    nl, rem = divmod(nwt, NUM_WORKERS)

    zrows = jnp.zeros((STRIPE, D), jnp.float32)
    zdeg = jnp.zeros((STRIPE,), jnp.float32)
    src1, dst1, zrows, zdeg = lax.optimization_barrier((src1, dst1, zrows, zdeg))
    p, dg = _make_sc(nl, rem)(x, src1, dst1, zrows, zdeg)

    d0 = dg[0].reshape(AGG_ROWS, 1)
    d1 = dg[1].reshape(AGG_ROWS, 1)

    grid = n // ROW_BLOCK
    out = pl.pallas_call(
        _fin_kernel,
        grid=(grid,),
        in_specs=[
            pl.BlockSpec((1, ROW_BLOCK, D), lambda i: (0, i, 0)),
            pl.BlockSpec((1, ROW_BLOCK, D), lambda i: (1, i, 0)),
            pl.BlockSpec((ROW_BLOCK, 1), lambda i: (i, 0)),
            pl.BlockSpec((ROW_BLOCK, 1), lambda i: (i, 0)),
            pl.BlockSpec((ROW_BLOCK, D), lambda i: (i, 0)),
            pl.BlockSpec((D, D), lambda i: (0, 0)),
            pl.BlockSpec((1, D), lambda i: (0, 0)),
        ],
        out_specs=pl.BlockSpec((ROW_BLOCK, D), lambda i: (i, 0)),
        out_shape=jax.ShapeDtypeStruct((n, D), jnp.float32),
    )(p, p, d0, d1, x, W1, b1.reshape(1, D))
    return out


# speedup vs baseline: 1.0525x; 1.0525x over previous
import functools

import jax
import jax.numpy as jnp
from jax import lax
from jax.experimental import pallas as pl
from jax.experimental.pallas import tpu as pltpu
from jax.experimental.pallas import tpu_sc as plsc

N_NODES = 10000
D = 128
ROW_BLOCK = 2000

NUM_CORES = 2
NUM_TILES = 16
NUM_WORKERS = NUM_CORES * NUM_TILES
WIN = 80
NBUF = 3
AGG_ROWS = 10240
STRIPE = AGG_ROWS // NUM_TILES


def _fin_kernel(p0_ref, p1_ref, d0_ref, d1_ref, x_ref, w_ref, b_ref, o_ref):
    deg_raw = d0_ref[...] + d1_ref[...]
    deg = jnp.maximum(deg_raw, 1.0)
    s = (p0_ref[...][0] + p1_ref[...][0]) / deg + x_ref[...]
    mm = jnp.dot(s, w_ref[...], preferred_element_type=jnp.float32)
    bfac = 1.0 + (deg_raw >= 0.5).astype(jnp.float32)
    o_ref[...] = jnp.maximum(mm + b_ref[...] * bfac, 0.0)


@functools.lru_cache(maxsize=None)
def _make_sc(nl, rem):
    assert nl >= 26
    UNROLL = 12
    k_end = (nl - 2) // UNROLL
    mesh = plsc.VectorSubcoreMesh(core_axis_name="c", subcore_axis_name="s")

    @functools.partial(
        pl.kernel,
        mesh=mesh,
        out_type=[
            jax.ShapeDtypeStruct((NUM_CORES, AGG_ROWS, D), jnp.float32),
            jax.ShapeDtypeStruct((NUM_CORES, AGG_ROWS), jnp.float32),
        ],
        scratch_types=[
            pltpu.VMEM((4, WIN), jnp.int32),
            pltpu.VMEM((4, WIN), jnp.int32),
            pltpu.VMEM((NBUF, WIN, D), jnp.float32),
            pltpu.VMEM((WIN,), jnp.float32),
            pltpu.VMEM_SHARED((AGG_ROWS, D), jnp.float32),
            pltpu.VMEM_SHARED((AGG_ROWS,), jnp.float32),
        ] + [pltpu.SemaphoreType.DMA] * (4 + 2 * NBUF + 1),
    )
    def sc(x_hbm, er_hbm, zrows_hbm, zdeg_hbm,
           agg_out, deg_out,
           sidx, didx, rows, ones, agg_sh, deg_sh,
           *sems):
        cid = lax.axis_index("c")
        tid = lax.axis_index("s")
        chunk = tid * NUM_CORES + cid
        isems = sems[0:4]
        rsems = sems[4:4 + NBUF]
        ssems = sems[4 + NBUF:4 + 2 * NBUF]
        zsem = sems[4 + 2 * NBUF]

        def wg(w):
            return chunk + NUM_WORKERS * w

        def i_start(w, s):
            pltpu.async_copy(er_hbm.at[0, wg(w)], sidx.at[s], isems[s])
            pltpu.async_copy(er_hbm.at[1, wg(w)], didx.at[s], isems[s])

        def i_wait(w, s):
            pltpu.make_async_copy(er_hbm.at[0, wg(w)], sidx.at[s], isems[s]).wait()
            pltpu.make_async_copy(er_hbm.at[1, wg(w)], didx.at[s], isems[s]).wait()

        def g_start(w, s, r):
            pltpu.async_copy(x_hbm.at[sidx.at[s]], rows.at[r], rsems[r])

        def g_wait(w, s, r):
            pltpu.make_async_copy(x_hbm.at[sidx.at[s]], rows.at[r], rsems[r]).wait()

        def s_start(w, s, r):
            pltpu.async_copy(rows.at[r], agg_sh.at[didx.at[s]], ssems[r], add=True)
            pltpu.async_copy(ones, deg_sh.at[didx.at[s]], ssems[r], add=True)

        def s_wait(w, s, r):
            pltpu.make_async_copy(rows.at[r], agg_sh.at[didx.at[s]], ssems[r]).wait()
            pltpu.make_async_copy(ones, deg_sh.at[didx.at[s]], ssems[r]).wait()

        def step(w, j):
            static = isinstance(w, int)
            if not static or w >= NBUF - 1:
                wp = w - (NBUF - 1)
                s_wait(wp, (j - (NBUF - 1)) % 4, (j - (NBUF - 1)) % NBUF)
            if not static or w + 2 < nl:
                i_start(w + 2, (j + 2) % 4)
            if not static or w + 1 < nl:
                i_wait(w + 1, (j + 1) % 4)
                g_start(w + 1, (j + 1) % 4, (j + 1) % NBUF)
            g_wait(w, j % 4, j % NBUF)
            s_start(w, j % 4, j % NBUF)

        for j in range(WIN // 16):
            ones[pl.ds(16 * j, 16)] = jnp.full((16,), 1.0, jnp.float32)

        zc1 = pltpu.async_copy(zrows_hbm, agg_sh.at[pl.ds(tid * STRIPE, STRIPE)], zsem)
        zc2 = pltpu.async_copy(zdeg_hbm, deg_sh.at[pl.ds(tid * STRIPE, STRIPE)], zsem)
        i_start(0, 0)
        i_start(1, 1)
        i_wait(0, 0)
        g_start(0, 0, 0)
        i_start(2, 2)
        i_wait(1, 1)
        g_start(1, 1, 1)
        g_wait(0, 0, 0)
        zc1.wait()
        zc2.wait()
        plsc.subcore_barrier()
        s_start(0, 0, 0)
        for w in range(1, UNROLL):
            step(w, w % UNROLL)

        def body(k, carry):
            w0 = UNROLL * k
            for j in range(UNROLL):
                step(w0 + j, j)
            return carry

        lax.fori_loop(1, k_end, body, 0)
        for w in range(UNROLL * k_end, nl):
            step(w, w % UNROLL)
        for w in range(nl - NBUF + 1, nl):
            s_wait(w, w % 4, w % NBUF)

        if rem:
            @pl.when(chunk < rem)
            def _():
                wr = NUM_WORKERS * nl + chunk
                pltpu.sync_copy(er_hbm.at[0, wr], sidx.at[0])
                pltpu.sync_copy(er_hbm.at[1, wr], didx.at[0])
                pltpu.sync_copy(x_hbm.at[sidx.at[0]], rows.at[0])
                pltpu.sync_copy(rows.at[0], agg_sh.at[didx.at[0]], add=True)
                pltpu.sync_copy(ones, deg_sh.at[didx.at[0]], add=True)

        plsc.subcore_barrier()
        pltpu.sync_copy(agg_sh.at[pl.ds(tid * STRIPE, STRIPE)],
                        agg_out.at[cid, pl.ds(tid * STRIPE, STRIPE)])
        pltpu.sync_copy(deg_sh.at[pl.ds(tid * STRIPE, STRIPE)],
                        deg_out.at[cid, pl.ds(tid * STRIPE, STRIPE)])

    return sc


def kernel(x, edge_index, W1, b1):
    n, d = x.shape
    e = edge_index.shape[1]
    ei = edge_index.astype(jnp.int32)
    if e % WIN:
        padn = WIN - e % WIN
        pidx = jnp.arange(padn, dtype=jnp.int32)
        ei = jnp.concatenate(
            [ei, jnp.stack([pidx % n, n + pidx % (AGG_ROWS - n)])], axis=1)
        e += padn
    nwt = e // WIN
    er = ei.reshape(2, nwt, WIN)
    nl, rem = divmod(nwt, NUM_WORKERS)

    zrows = jnp.zeros((STRIPE, D), jnp.float32)
    zdeg = jnp.zeros((STRIPE,), jnp.float32)
    er, zrows, zdeg = lax.optimization_barrier((er, zrows, zdeg))
    p, dg = _make_sc(nl, rem)(x, er, zrows, zdeg)

    d0 = dg[0].reshape(AGG_ROWS, 1)
    d1 = dg[1].reshape(AGG_ROWS, 1)

    grid = n // ROW_BLOCK
    out = pl.pallas_call(
        _fin_kernel,
        grid=(grid,),
        in_specs=[
            pl.BlockSpec((1, ROW_BLOCK, D), lambda i: (0, i, 0)),
            pl.BlockSpec((1, ROW_BLOCK, D), lambda i: (1, i, 0)),
            pl.BlockSpec((ROW_BLOCK, 1), lambda i: (i, 0)),
            pl.BlockSpec((ROW_BLOCK, 1), lambda i: (i, 0)),
            pl.BlockSpec((ROW_BLOCK, D), lambda i: (i, 0)),
            pl.BlockSpec((D, D), lambda i: (0, 0)),
            pl.BlockSpec((1, D), lambda i: (0, 0)),
        ],
        out_specs=pl.BlockSpec((ROW_BLOCK, D), lambda i: (i, 0)),
        out_shape=jax.ShapeDtypeStruct((n, D), jnp.float32),
    )(p, p, d0, d1, x, W1, b1.reshape(1, D))
    return out
